# baseline (speedup 1.0000x reference)
import functools

import jax
import jax.numpy as jnp
from jax import lax
from jax.experimental import pallas as pl
from jax.experimental.pallas import tpu as pltpu
from jax.experimental.pallas import tpu_sc as plsc

_B = 16384
_V = 1000000
_D = 16
_K = 2
_F = 1
_H = 64
_EPS = 1e-06
_TOT = _K * (1 + 2 * _F)
_P = 2 * _TOT

_NC, _NS, _L = 2, 16, 16
_NW = _NC * _NS

_CW = 2048
_NCHUNK = 489
_TAILW = 512
_TBASE = 488 * _CW + _TAILW
_WCAP = 768
_CCAP = 96
_NWV = _WCAP // _L
_NCV = _CCAP // _L
_DUMP = _B


def _iota():
    return lax.iota(jnp.int32, _L)


def _full(v):
    return jnp.full((_L,), v, jnp.int32)


@functools.cache
def _get_sc_gather():
    mesh = plsc.VectorSubcoreMesh(core_axis_name="c", subcore_axis_name="s")

    @functools.partial(
        pl.kernel,
        mesh=mesh,
        compiler_params=pltpu.CompilerParams(needs_layout_passes=False),
        out_type=jax.ShapeDtypeStruct((_B + 8, 128), jnp.float32),
        scratch_types=[
            pltpu.VMEM((_B,), jnp.int32),
            pltpu.VMEM((_B,), jnp.float32),
            pltpu.VMEM((16, _CW), jnp.float32),
            pltpu.VMEM((_WCAP,), jnp.int32),
            pltpu.VMEM((_WCAP,), jnp.int32),
            pltpu.VMEM((_CCAP + 96,), jnp.int32),
            pltpu.VMEM((_CCAP + 96,), jnp.int32),
            pltpu.VMEM((_CCAP + 96,), jnp.int32),
            pltpu.VMEM((1, _CCAP), jnp.int32),
            pltpu.VMEM((_CCAP, 128), jnp.float32),
            pltpu.SemaphoreType.DMA,
        ],
    )
    def _sc_gather(emb_hbm, idx_hbm, u_hbm, out_hbm, idx_all, u_all, staged,
                   want_i, want_p, ch_s, ch_p, ch_i, posrow, rows, sem):
        wid = lax.axis_index("s") * _NC + lax.axis_index("c")
        dump = _DUMP + (wid & 7)

        pltpu.sync_copy(idx_hbm, idx_all)
        pltpu.sync_copy(u_hbm, u_all)

        for k in range(_NWV):
            want_i[pl.ds(k * _L, _L)] = _full(1 << 29)
            want_p[pl.ds(k * _L, _L)] = _full(dump)

        def p1(k, cnt):
            iv = idx_all[pl.ds(k * _L, _L)]
            m = (lax.shift_right_logical(iv, 11) & 31) == wid
            mi = m.astype(jnp.int32)
            dst = cnt + _iota()
            plsc.store_scatter(want_i, [dst], iv, mask=m)
            plsc.store_scatter(want_p, [dst], k * _L + _iota(), mask=m)
            return cnt + 16

        lax.fori_loop(0, _B // _L, p1, jnp.int32(0))

        def do_chunk(c, width):
            for k in range(_NCV):
                ch_s[pl.ds(k * _L, _L)] = _full(0)
                ch_p[pl.ds(k * _L, _L)] = _full(dump)
                ch_i[pl.ds(k * _L, _L)] = _full(0)

            def p2(k, cc):
                wv = want_i[pl.ds(k * _L, _L)]
                pv = want_p[pl.ds(k * _L, _L)]
                m = lax.shift_right_logical(wv, 11) == c
                mi = m.astype(jnp.int32)
                dst = cc + _iota()
                plsc.store_scatter(ch_s, [dst], wv & 2047, mask=m)
                plsc.store_scatter(ch_p, [dst], pv, mask=m)
                plsc.store_scatter(ch_i, [dst], wv, mask=m)
                return cc + 16

            lax.fori_loop(0, _NWV, p2, jnp.int32(0))


            for v in range(_NCV):
                svec = jnp.minimum(ch_s[pl.ds(v * _L, _L)], width - 1)
                pvec = ch_p[pl.ds(v * _L, _L)]
                dvec = _full(v * _L) + _iota()
                for f in range(_D):
                    vf = plsc.load_gather(staged, [_full(f), svec])
                    plsc.store_scatter(rows, [dvec, _full(f)], vf)
                uv = plsc.load_gather(u_all, [jnp.minimum(pvec, _B - 1)])
                plsc.store_scatter(rows, [dvec, _full(_D)], uv)
                iv = ch_i[pl.ds(v * _L, _L)].astype(jnp.float32)
                plsc.store_scatter(rows, [dvec, _full(_D + 1)], iv)
                posrow[0, pl.ds(v * _L, _L)] = pvec

            pltpu.async_copy(rows, out_hbm.at[posrow.at[0]], sem).wait()

        def main_body(j, carry):
            do_chunk(wid + 32 * j, _CW)
            return carry

        lax.fori_loop(0, 15, main_body, jnp.int32(0))

        @pl.when(wid < 8)
        def _():
            do_chunk(480 + wid, _CW)

        @pl.when(wid == 8)
        def _():
            do_chunk(488, _TAILW)

    return _sc_gather


_BM = 2048


def _tc_body(g_ref, tail_ref, w1_ref, uw_ref, b1_ref, w2_ref, b2_ref,
             w3t_ref, b3_ref, out_ref):
    g = g_ref[...]
    x = g[:, :_D]
    u = g[:, _D:_D + 1]
    sv = g[:, _D + 1:_D + 2]
    delta = sv - float(_TBASE)
    onehot = (delta == lax.broadcasted_iota(jnp.int32, (1, 64), 1
                                            ).astype(jnp.float32)
              ).astype(jnp.float32)
    x_fix = jnp.dot(onehot, tail_ref[...], preferred_element_type=jnp.float32)
    x = jnp.where(delta >= 0.0, x_fix, x)
    h = jnp.dot(x, w1_ref[...], preferred_element_type=jnp.float32)
    h = jnp.maximum(h + b1_ref[...] + u * uw_ref[...], 0.0)
    h = jnp.dot(h, w2_ref[...], preferred_element_type=jnp.float32)
    h = jnp.maximum(h + b2_ref[...], 0.0)
    phi = lax.dot_general(w3t_ref[...], h, (((1,), (1,)), ((), ())),
                          preferred_element_type=jnp.float32)
    phi = phi + b3_ref[...]
    row = lax.broadcasted_iota(jnp.int32, phi.shape, 0)
    is_scale = ((row >= 4) & (row < 6)) | (row >= 10)
    sp = jnp.maximum(phi, 0.0) + jnp.log1p(jnp.exp(-jnp.abs(phi))) + _EPS
    out_ref[...] = jnp.where(is_scale, sp, phi)


_tc_mlp = pl.pallas_call(
    _tc_body,
    grid=(_B // _BM,),
    in_specs=[
        pl.BlockSpec((_BM, 128), lambda i: (i, 0)),
        pl.BlockSpec((64, _D), lambda i: (0, 0)),
        pl.BlockSpec((_D, 2 * _H), lambda i: (0, 0)),
        pl.BlockSpec((1, 2 * _H), lambda i: (0, 0)),
        pl.BlockSpec((1, 2 * _H), lambda i: (0, 0)),
        pl.BlockSpec((2 * _H, 2 * _H), lambda i: (0, 0)),
        pl.BlockSpec((1, 2 * _H), lambda i: (0, 0)),
        pl.BlockSpec((_P, 2 * _H), lambda i: (0, 0)),
        pl.BlockSpec((_P, 1), lambda i: (0, 0)),
    ],
    out_specs=pl.BlockSpec((_P, _BM), lambda i: (0, i)),
    out_shape=jax.ShapeDtypeStruct((_P, _B), jnp.float32),
)


def kernel(source, upstream_speed, emb, uW1, ub1, uW2, ub2, uW3, ub3,
           dW1, db1, dW2, db2, dW3, db3):
    src = source.astype(jnp.int32)
    gath = _get_sc_gather()(emb.T, src, upstream_speed)
    emb_tail = lax.slice(emb, (_TBASE, 0), (_V, _D))

    zhh = jnp.zeros((_H, _H), jnp.float32)
    zph = jnp.zeros((_TOT, _H), jnp.float32)
    w1c = jnp.concatenate([uW1, dW1[:_D]], axis=1)
    uw = jnp.concatenate([jnp.zeros((_H,), jnp.float32), dW1[_D]])[None, :]
    b1c = jnp.concatenate([ub1, db1])[None, :]
    w2c = jnp.concatenate(
        [jnp.concatenate([uW2, zhh], axis=1),
         jnp.concatenate([zhh, dW2], axis=1)], axis=0)
    b2c = jnp.concatenate([ub2, db2])[None, :]
    w3t = jnp.concatenate(
        [jnp.concatenate([uW3.T, zph], axis=1),
         jnp.concatenate([zph, dW3.T], axis=1)], axis=0)
    b3c = jnp.concatenate([ub3, db3])[:, None]

    phi = _tc_mlp(gath, emb_tail, w1c, uw, b1c, w2c, b2c, w3t, b3c)

    up_logits = phi[0:2].T
    up_loc = phi[2:4].T.reshape(_B, _K, _F)
    up_scale = phi[4:6].T.reshape(_B, _K, _F)
    down_logits = phi[6:8].T
    down_loc = phi[8:10].T.reshape(_B, _K, _F)
    down_scale = phi[10:12].T.reshape(_B, _K, _F)
    return (up_logits, up_loc, up_scale, down_logits, down_loc, down_scale)


# speedup vs baseline: 2.1668x; 2.1668x over previous
import functools

import jax
import jax.numpy as jnp
from jax import lax
from jax.experimental import pallas as pl
from jax.experimental.pallas import tpu as pltpu
from jax.experimental.pallas import tpu_sc as plsc

_B = 16384
_V = 1000000
_D = 16
_K = 2
_F = 1
_H = 64
_EPS = 1e-06
_TOT = _K * (1 + 2 * _F)
_P = 2 * _TOT

_NC, _NS, _L = 2, 16, 16
_NW = _NC * _NS

_CW = 2048
_NCHUNK = 489
_TAILW = 512
_TBASE = 488 * _CW + _TAILW
_WCAP = 768
_CCAP = 96
_NWV = _WCAP // _L
_NCV = _CCAP // _L
_NPAD = 2048


def _iota():
    return lax.iota(jnp.int32, _L)


def _full(v):
    return jnp.full((_L,), v, jnp.int32)


@functools.cache
def _get_sc_gather():
    mesh = plsc.VectorSubcoreMesh(core_axis_name="c", subcore_axis_name="s")

    @functools.partial(
        pl.kernel,
        mesh=mesh,
        compiler_params=pltpu.CompilerParams(needs_layout_passes=False),
        out_type=jax.ShapeDtypeStruct((_B + _NPAD, 128), jnp.float32),
        scratch_types=[
            pltpu.VMEM((_B,), jnp.int32),
            pltpu.VMEM((_B,), jnp.float32),
            pltpu.VMEM((16, _CW), jnp.float32),
            pltpu.VMEM((_WCAP,), jnp.int32),
            pltpu.VMEM((_CCAP + 96,), jnp.int32),
            pltpu.VMEM((1, _CCAP), jnp.int32),
            pltpu.VMEM((_CCAP, 128), jnp.float32),
            pltpu.SemaphoreType.DMA,
        ],
    )
    def _sc_gather(emb_hbm, idx_hbm, u_hbm, out_hbm, idx_all, u_all, staged,
                   want, chb, posrow, rows, sem):
        wid = lax.axis_index("s") * _NC + lax.axis_index("c")

        pltpu.sync_copy(idx_hbm, idx_all)
        pltpu.sync_copy(u_hbm, u_all)

        for k in range(_NWV):
            want[pl.ds(k * _L, _L)] = _full(1 << 30)

        def p1(k, cnt):
            for uu in range(4):
                kk = k * 4 + uu
                iv = idx_all[pl.ds(kk * _L, _L)]
                m = (lax.shift_right_logical(iv, 11) & 31) == wid
                mi = m.astype(jnp.int32)
                packed = (lax.shift_left(lax.shift_right_logical(iv, 16), 26)
                          | lax.shift_left(iv & 2047, 15)
                          | (kk * _L + _iota()))
                dst = cnt + plsc.cumsum(mi) - 1
                plsc.store_scatter(want, [dst], packed, mask=m)
                cnt = cnt + jnp.sum(mi)
            return cnt

        lax.fori_loop(0, _B // _L // 4, p1, jnp.int32(0))

        def do_chunk(c, j, width):
            for k in range(_NCV):
                pad = _B + ((wid * 61 + c * 7 + k * 13) & (_NPAD - 1))
                chb[pl.ds(k * _L, _L)] = _iota() * 0 + pad

            def p2(k, cc):
                for uu in range(4):
                    kk = k * 4 + uu
                    wv = want[pl.ds(kk * _L, _L)]
                    m = lax.shift_right_logical(wv, 26) == j
                    mi = m.astype(jnp.int32)
                    dst = cc + plsc.cumsum(mi) - 1
                    plsc.store_scatter(chb, [dst], wv, mask=m)
                    cc = cc + jnp.sum(mi)
                return cc

            lax.fori_loop(0, _NWV // 4, p2, jnp.int32(0))

            pltpu.sync_copy(
                emb_hbm.at[:, pl.ds(c * _CW, width)], staged.at[:, :width])

            for v in range(_NCV):
                wv = chb[pl.ds(v * _L, _L)]
                pvec = wv & 32767
                sun = lax.shift_right_logical(wv, 15) & 2047
                svec = jnp.minimum(sun, width - 1)
                dvec = _full(v * _L) + _iota()
                for f in range(_D):
                    vf = plsc.load_gather(staged, [_full(f), svec])
                    plsc.store_scatter(rows, [dvec, _full(f)], vf)
                uv = plsc.load_gather(u_all, [jnp.minimum(pvec, _B - 1)])
                plsc.store_scatter(rows, [dvec, _full(_D)], uv)
                ivf = (c * _CW + sun).astype(jnp.float32)
                plsc.store_scatter(rows, [dvec, _full(_D + 1)], ivf)
                posrow[0, pl.ds(v * _L, _L)] = pvec

            pltpu.async_copy(rows, out_hbm.at[posrow.at[0]], sem).wait()

        def main_body(j, carry):
            do_chunk(wid + 32 * j, j, _CW)
            return carry

        lax.fori_loop(0, 15, main_body, jnp.int32(0))

        @pl.when(wid < 8)
        def _():
            do_chunk(480 + wid, 15, _CW)

        @pl.when(wid == 8)
        def _():
            do_chunk(488, 15, _TAILW)

    return _sc_gather


_BM = 2048


def _tc_body(g_ref, tail_ref, w1_ref, uw_ref, b1_ref, w2_ref, b2_ref,
             w3t_ref, b3_ref, out_ref):
    g = g_ref[...]
    x = g[:, :_D]
    u = g[:, _D:_D + 1]
    sv = g[:, _D + 1:_D + 2]
    delta = sv - float(_TBASE)
    onehot = (delta == lax.broadcasted_iota(jnp.int32, (1, 64), 1
                                            ).astype(jnp.float32)
              ).astype(jnp.float32)
    x_fix = jnp.dot(onehot, tail_ref[...], preferred_element_type=jnp.float32)
    x = jnp.where(delta >= 0.0, x_fix, x)
    h = jnp.dot(x, w1_ref[...], preferred_element_type=jnp.float32)
    h = jnp.maximum(h + b1_ref[...] + u * uw_ref[...], 0.0)
    h = jnp.dot(h, w2_ref[...], preferred_element_type=jnp.float32)
    h = jnp.maximum(h + b2_ref[...], 0.0)
    phi = lax.dot_general(w3t_ref[...], h, (((1,), (1,)), ((), ())),
                          preferred_element_type=jnp.float32)
    phi = phi + b3_ref[...]
    row = lax.broadcasted_iota(jnp.int32, phi.shape, 0)
    is_scale = ((row >= 4) & (row < 6)) | (row >= 10)
    sp = jnp.maximum(phi, 0.0) + jnp.log1p(jnp.exp(-jnp.abs(phi))) + _EPS
    out_ref[...] = jnp.where(is_scale, sp, phi)


_tc_mlp = pl.pallas_call(
    _tc_body,
    grid=(_B // _BM,),
    in_specs=[
        pl.BlockSpec((_BM, 128), lambda i: (i, 0)),
        pl.BlockSpec((64, _D), lambda i: (0, 0)),
        pl.BlockSpec((_D, 2 * _H), lambda i: (0, 0)),
        pl.BlockSpec((1, 2 * _H), lambda i: (0, 0)),
        pl.BlockSpec((1, 2 * _H), lambda i: (0, 0)),
        pl.BlockSpec((2 * _H, 2 * _H), lambda i: (0, 0)),
        pl.BlockSpec((1, 2 * _H), lambda i: (0, 0)),
        pl.BlockSpec((_P, 2 * _H), lambda i: (0, 0)),
        pl.BlockSpec((_P, 1), lambda i: (0, 0)),
    ],
    out_specs=pl.BlockSpec((_P, _BM), lambda i: (0, i)),
    out_shape=jax.ShapeDtypeStruct((_P, _B), jnp.float32),
)


def kernel(source, upstream_speed, emb, uW1, ub1, uW2, ub2, uW3, ub3,
           dW1, db1, dW2, db2, dW3, db3):
    src = source.astype(jnp.int32)
    gath = _get_sc_gather()(emb.T, src, upstream_speed)
    emb_tail = lax.slice(emb, (_TBASE, 0), (_V, _D))

    zhh = jnp.zeros((_H, _H), jnp.float32)
    zph = jnp.zeros((_TOT, _H), jnp.float32)
    w1c = jnp.concatenate([uW1, dW1[:_D]], axis=1)
    uw = jnp.concatenate([jnp.zeros((_H,), jnp.float32), dW1[_D]])[None, :]
    b1c = jnp.concatenate([ub1, db1])[None, :]
    w2c = jnp.concatenate(
        [jnp.concatenate([uW2, zhh], axis=1),
         jnp.concatenate([zhh, dW2], axis=1)], axis=0)
    b2c = jnp.concatenate([ub2, db2])[None, :]
    w3t = jnp.concatenate(
        [jnp.concatenate([uW3.T, zph], axis=1),
         jnp.concatenate([zph, dW3.T], axis=1)], axis=0)
    b3c = jnp.concatenate([ub3, db3])[:, None]

    phi = _tc_mlp(gath, emb_tail, w1c, uw, b1c, w2c, b2c, w3t, b3c)

    up_logits = phi[0:2].T
    up_loc = phi[2:4].T.reshape(_B, _K, _F)
    up_scale = phi[4:6].T.reshape(_B, _K, _F)
    down_logits = phi[6:8].T
    down_loc = phi[8:10].T.reshape(_B, _K, _F)
    down_scale = phi[10:12].T.reshape(_B, _K, _F)
    return (up_logits, up_loc, up_scale, down_logits, down_loc, down_scale)


# speedup vs baseline: 2.4902x; 1.1493x over previous
import functools

import jax
import jax.numpy as jnp
from jax import lax
from jax.experimental import pallas as pl
from jax.experimental.pallas import tpu as pltpu
from jax.experimental.pallas import tpu_sc as plsc

_B = 16384
_V = 1000000
_D = 16
_K = 2
_F = 1
_H = 64
_EPS = 1e-06
_TOT = _K * (1 + 2 * _F)
_P = 2 * _TOT

_NC, _NS, _L = 2, 16, 16
_NW = _NC * _NS

_CW = 2048
_NCHUNK = 489
_TAILW = 512
_TBASE = 488 * _CW + _TAILW
_WCAP = 768
_CCAP = 96
_NWV = _WCAP // _L
_NCV = _CCAP // _L
_NPAD = 2048


def _iota():
    return lax.iota(jnp.int32, _L)


def _full(v):
    return jnp.full((_L,), v, jnp.int32)


@functools.cache
def _get_sc_gather():
    mesh = plsc.VectorSubcoreMesh(core_axis_name="c", subcore_axis_name="s")

    @functools.partial(
        pl.kernel,
        mesh=mesh,
        compiler_params=pltpu.CompilerParams(needs_layout_passes=False),
        out_type=jax.ShapeDtypeStruct((_B + _NPAD, 128), jnp.float32),
        scratch_types=[
            pltpu.VMEM((_B,), jnp.int32),
            pltpu.VMEM((_B,), jnp.float32),
            pltpu.VMEM((16, _CW), jnp.float32),
            pltpu.VMEM((16, _CW), jnp.float32),
            pltpu.VMEM((_WCAP,), jnp.int32),
            pltpu.VMEM((_CCAP + 96,), jnp.int32),
            pltpu.VMEM((1, _CCAP), jnp.int32),
            pltpu.VMEM((_CCAP, 128), jnp.float32),
            pltpu.SemaphoreType.DMA,
            pltpu.SemaphoreType.DMA,
            pltpu.SemaphoreType.DMA,
        ],
    )
    def _sc_gather(emb_hbm, idx_hbm, u_hbm, out_hbm, idx_all, u_all, st_a,
                   st_b, want, chb, posrow, rows, sem, sem_a, sem_b):
        wid = lax.axis_index("s") * _NC + lax.axis_index("c")

        pltpu.sync_copy(idx_hbm, idx_all)
        pltpu.sync_copy(u_hbm, u_all)

        for k in range(_NWV):
            want[pl.ds(k * _L, _L)] = _full(1 << 30)

        def p1(k, cnt):
            for uu in range(4):
                kk = k * 4 + uu
                iv = idx_all[pl.ds(kk * _L, _L)]
                m = (lax.shift_right_logical(iv, 11) & 31) == wid
                mi = m.astype(jnp.int32)
                packed = (lax.shift_left(lax.shift_right_logical(iv, 16), 26)
                          | lax.shift_left(iv & 2047, 15)
                          | (kk * _L + _iota()))
                dst = cnt + plsc.cumsum(mi) - 1
                plsc.store_scatter(want, [dst], packed, mask=m)
                cnt = cnt + jnp.sum(mi)
            return cnt

        lax.fori_loop(0, _B // _L // 4, p1, jnp.int32(0))

        def start(c, buf, sm):
            pltpu.async_copy(emb_hbm.at[:, pl.ds(c * _CW, _CW)], buf, sm)

        def wait(c, buf, sm):
            pltpu.make_async_copy(
                emb_hbm.at[:, pl.ds(c * _CW, _CW)], buf, sm).wait()

        def p2_fill(c, j):
            for k in range(_NCV):
                pad = _B + ((wid * 61 + c * 7 + k * 13) & (_NPAD - 1))
                chb[pl.ds(k * _L, _L)] = _iota() * 0 + pad

            def p2(k, cc):
                for uu in range(4):
                    kk = k * 4 + uu
                    wv = want[pl.ds(kk * _L, _L)]
                    m = lax.shift_right_logical(wv, 26) == j
                    mi = m.astype(jnp.int32)
                    dst = cc + plsc.cumsum(mi) - 1
                    plsc.store_scatter(chb, [dst], wv, mask=m)
                    cc = cc + jnp.sum(mi)
                return cc

            lax.fori_loop(0, _NWV // 4, p2, jnp.int32(0))

        def extract(c, width, buf):
            for v in range(_NCV):
                wv = chb[pl.ds(v * _L, _L)]
                pvec = wv & 32767
                sun = lax.shift_right_logical(wv, 15) & 2047
                svec = jnp.minimum(sun, width - 1)
                dvec = _full(v * _L) + _iota()
                for f in range(_D):
                    vf = plsc.load_gather(buf, [_full(f), svec])
                    plsc.store_scatter(rows, [dvec, _full(f)], vf)
                uv = plsc.load_gather(u_all, [jnp.minimum(pvec, _B - 1)])
                plsc.store_scatter(rows, [dvec, _full(_D)], uv)
                ivf = (c * _CW + sun).astype(jnp.float32)
                plsc.store_scatter(rows, [dvec, _full(_D + 1)], ivf)
                posrow[0, pl.ds(v * _L, _L)] = pvec

            pltpu.async_copy(rows, out_hbm.at[posrow.at[0]], sem).wait()

        start(wid, st_a, sem_a)

        def pair(t, carry):
            j0 = 2 * t
            c0 = wid + 32 * j0
            c1 = c0 + 32
            start(c1, st_b, sem_b)
            p2_fill(c0, j0)
            wait(c0, st_a, sem_a)
            extract(c0, _CW, st_a)
            start(c0 + 64, st_a, sem_a)
            p2_fill(c1, j0 + 1)
            wait(c1, st_b, sem_b)
            extract(c1, _CW, st_b)
            return carry

        lax.fori_loop(0, 7, pair, jnp.int32(0))

        c14 = wid + 32 * 14
        p2_fill(c14, 14)
        wait(c14, st_a, sem_a)
        extract(c14, _CW, st_a)

        @pl.when(wid < 8)
        def _():
            c15 = 480 + wid
            p2_fill(c15, 15)
            pltpu.sync_copy(emb_hbm.at[:, pl.ds(c15 * _CW, _CW)], st_a)
            extract(c15, _CW, st_a)

        @pl.when(wid == 8)
        def _():
            p2_fill(488, 15)
            pltpu.sync_copy(
                emb_hbm.at[:, pl.ds(488 * _CW, _TAILW)],
                st_a.at[:, :_TAILW])
            extract(488, _TAILW, st_a)

    return _sc_gather


_BM = 2048


def _tc_body(g_ref, tail_ref, w1_ref, uw_ref, b1_ref, w2_ref, b2_ref,
             w3t_ref, b3_ref, out_ref):
    g = g_ref[...]
    x = g[:, :_D]
    u = g[:, _D:_D + 1]
    sv = g[:, _D + 1:_D + 2]
    delta = sv - float(_TBASE)
    onehot = (delta == lax.broadcasted_iota(jnp.int32, (1, 64), 1
                                            ).astype(jnp.float32)
              ).astype(jnp.float32)
    x_fix = jnp.dot(onehot, tail_ref[...], preferred_element_type=jnp.float32)
    x = jnp.where(delta >= 0.0, x_fix, x)
    h = jnp.dot(x, w1_ref[...], preferred_element_type=jnp.float32)
    h = jnp.maximum(h + b1_ref[...] + u * uw_ref[...], 0.0)
    h = jnp.dot(h, w2_ref[...], preferred_element_type=jnp.float32)
    h = jnp.maximum(h + b2_ref[...], 0.0)
    phi = lax.dot_general(w3t_ref[...], h, (((1,), (1,)), ((), ())),
                          preferred_element_type=jnp.float32)
    phi = phi + b3_ref[...]
    row = lax.broadcasted_iota(jnp.int32, phi.shape, 0)
    is_scale = ((row >= 4) & (row < 6)) | (row >= 10)
    sp = jnp.maximum(phi, 0.0) + jnp.log1p(jnp.exp(-jnp.abs(phi))) + _EPS
    out_ref[...] = jnp.where(is_scale, sp, phi)


_tc_mlp = pl.pallas_call(
    _tc_body,
    grid=(_B // _BM,),
    in_specs=[
        pl.BlockSpec((_BM, 128), lambda i: (i, 0)),
        pl.BlockSpec((64, _D), lambda i: (0, 0)),
        pl.BlockSpec((_D, 2 * _H), lambda i: (0, 0)),
        pl.BlockSpec((1, 2 * _H), lambda i: (0, 0)),
        pl.BlockSpec((1, 2 * _H), lambda i: (0, 0)),
        pl.BlockSpec((2 * _H, 2 * _H), lambda i: (0, 0)),
        pl.BlockSpec((1, 2 * _H), lambda i: (0, 0)),
        pl.BlockSpec((_P, 2 * _H), lambda i: (0, 0)),
        pl.BlockSpec((_P, 1), lambda i: (0, 0)),
    ],
    out_specs=pl.BlockSpec((_P, _BM), lambda i: (0, i)),
    out_shape=jax.ShapeDtypeStruct((_P, _B), jnp.float32),
)


def kernel(source, upstream_speed, emb, uW1, ub1, uW2, ub2, uW3, ub3,
           dW1, db1, dW2, db2, dW3, db3):
    src = source.astype(jnp.int32)
    gath = _get_sc_gather()(emb.T, src, upstream_speed)
    emb_tail = lax.slice(emb, (_TBASE, 0), (_V, _D))

    zhh = jnp.zeros((_H, _H), jnp.float32)
    zph = jnp.zeros((_TOT, _H), jnp.float32)
    w1c = jnp.concatenate([uW1, dW1[:_D]], axis=1)
    uw = jnp.concatenate([jnp.zeros((_H,), jnp.float32), dW1[_D]])[None, :]
    b1c = jnp.concatenate([ub1, db1])[None, :]
    w2c = jnp.concatenate(
        [jnp.concatenate([uW2, zhh], axis=1),
         jnp.concatenate([zhh, dW2], axis=1)], axis=0)
    b2c = jnp.concatenate([ub2, db2])[None, :]
    w3t = jnp.concatenate(
        [jnp.concatenate([uW3.T, zph], axis=1),
         jnp.concatenate([zph, dW3.T], axis=1)], axis=0)
    b3c = jnp.concatenate([ub3, db3])[:, None]

    phi = _tc_mlp(gath, emb_tail, w1c, uw, b1c, w2c, b2c, w3t, b3c)

    up_logits = phi[0:2].T
    up_loc = phi[2:4].T.reshape(_B, _K, _F)
    up_scale = phi[4:6].T.reshape(_B, _K, _F)
    down_logits = phi[6:8].T
    down_loc = phi[8:10].T.reshape(_B, _K, _F)
    down_scale = phi[10:12].T.reshape(_B, _K, _F)
    return (up_logits, up_loc, up_scale, down_logits, down_loc, down_scale)
